# Initial kernel scaffold; baseline (speedup 1.0000x reference)
import jax
import jax.numpy as jnp
from jax.experimental import pallas as pl


def kernel(features, edge_index, W1, b1, W2, b2, W3, b3):
    raise NotImplementedError("write your pallas kernel here")


# speedup vs baseline: 1.2269x; 1.2269x over previous
import functools

import jax
import jax.numpy as jnp
from jax import lax
from jax.experimental import pallas as pl
from jax.experimental.pallas import tpu as pltpu
from jax.experimental.pallas import tpu_sc as plsc

N = 10000
NP = 10240
E = 50000
EP = 51200
W = 128
EB = 128
K1 = 1440
H1 = 1024
H2 = 512
DO = 128


def _mm1_body(a_ref, w_ref, o_ref):
    o_ref[0] = jnp.dot(a_ref[...], w_ref[...],
                       preferred_element_type=jnp.float32)


def _mm1(a, w):
    c = H1 // W
    return pl.pallas_call(
        _mm1_body,
        grid=(NP // 256, c),
        in_specs=[
            pl.BlockSpec((256, K1), lambda i, j: (i, 0)),
            pl.BlockSpec((K1, W), lambda i, j: (0, j)),
        ],
        out_specs=pl.BlockSpec((1, 256, W), lambda i, j: (j, i, 0)),
        out_shape=jax.ShapeDtypeStruct((c, NP, W), jnp.float32),
    )(a, w)


def _mm2_body(a_ref, b_ref, w_ref, o_ref):
    k = pl.program_id(2)
    x = jnp.maximum(a_ref[0] + b_ref[0], 0.0)
    y = jnp.dot(x, w_ref[...], preferred_element_type=jnp.float32)

    @pl.when(k == 0)
    def _():
        o_ref[0] = y

    @pl.when(k > 0)
    def _():
        o_ref[0] += y


def _mm2(agg, b, w):
    ci = agg.shape[0]
    co = w.shape[1] // W
    return pl.pallas_call(
        _mm2_body,
        grid=(NP // 256, co, ci),
        in_specs=[
            pl.BlockSpec((1, 256, W), lambda i, j, k: (k, i, 0)),
            pl.BlockSpec((1, 1, W), lambda i, j, k: (k, 0, 0)),
            pl.BlockSpec((W, W), lambda i, j, k: (k, j)),
        ],
        out_specs=pl.BlockSpec((1, 256, W), lambda i, j, k: (j, i, 0)),
        out_shape=jax.ShapeDtypeStruct((co, NP, W), jnp.float32),
        compiler_params=pltpu.CompilerParams(
            dimension_semantics=("parallel", "parallel", "arbitrary")),
    )(agg, b, w)


def _mm3_body(a_ref, b2_ref, w_ref, b3_ref, o_ref, acc_ref):
    k = pl.program_id(1)
    x = jnp.maximum(a_ref[0] + b2_ref[0], 0.0)
    y = jnp.dot(x, w_ref[...], preferred_element_type=jnp.float32)

    @pl.when(k == 0)
    def _():
        acc_ref[...] = y

    @pl.when(k > 0)
    def _():
        acc_ref[...] += y

    @pl.when(k == pl.num_programs(1) - 1)
    def _():
        o_ref[...] = jnp.maximum(acc_ref[...] + b3_ref[...], 0.0)


def _mm3(agg, b2, w, b3):
    ci = agg.shape[0]
    return pl.pallas_call(
        _mm3_body,
        grid=(NP // 256, ci),
        in_specs=[
            pl.BlockSpec((1, 256, W), lambda i, k: (k, i, 0)),
            pl.BlockSpec((1, 1, W), lambda i, k: (k, 0, 0)),
            pl.BlockSpec((W, DO), lambda i, k: (k, 0)),
            pl.BlockSpec((1, DO), lambda i, k: (0, 0)),
        ],
        out_specs=pl.BlockSpec((256, DO), lambda i, k: (i, 0)),
        out_shape=jax.ShapeDtypeStruct((NP, DO), jnp.float32),
        scratch_shapes=[pltpu.VMEM((256, DO), jnp.float32)],
        compiler_params=pltpu.CompilerParams(
            dimension_semantics=("parallel", "arbitrary")),
    )(agg, b2, w, b3)


def _make_agg(c_chunks):
    half = c_chunks // 2
    rows_t = NP // 16
    ept = EP // 16
    nb = ept // EB
    mesh = plsc.VectorSubcoreMesh(core_axis_name="c", subcore_axis_name="s")

    @functools.partial(
        pl.kernel,
        mesh=mesh,
        out_type=jax.ShapeDtypeStruct((c_chunks, NP, W), jnp.float32),
        scratch_types=[
            pltpu.VMEM_SHARED((NP, W), jnp.float32),
            pltpu.VMEM((EB,), jnp.int32),
            pltpu.VMEM((EB,), jnp.int32),
            pltpu.VMEM((EB, W), jnp.float32),
            pltpu.SemaphoreType.DMA,
        ],
    )
    def agg(h, src, dst, zeros, out, slab, src_v, dst_v, rows_v, sem):
        cid = lax.axis_index("c")
        sid = lax.axis_index("s")
        r0 = sid * rows_t
        e0 = sid * ept

        def chunk_body(ci, carry):
            chunk = cid * half + ci
            pltpu.sync_copy(zeros.at[pl.ds(r0, rows_t)],
                            slab.at[pl.ds(r0, rows_t)])
            plsc.subcore_barrier()

            def batch_body(b, carry2):
                eb = e0 + b * EB
                pltpu.sync_copy(src.at[pl.ds(eb, EB)], src_v)
                pltpu.sync_copy(dst.at[pl.ds(eb, EB)], dst_v)
                pltpu.async_copy(h.at[chunk].at[src_v], rows_v, sem).wait()
                pltpu.sync_copy(rows_v, slab.at[dst_v], add=True)
                return carry2

            lax.fori_loop(0, nb, batch_body, 0)
            plsc.subcore_barrier()
            pltpu.sync_copy(slab.at[pl.ds(r0, rows_t)],
                            out.at[chunk, pl.ds(r0, rows_t)])
            plsc.subcore_barrier()
            return carry

        lax.fori_loop(0, half, chunk_body, 0)

    return agg


_agg8 = _make_agg(8)
_agg4 = _make_agg(4)


def kernel(features, edge_index, W1, b1, W2, b2, W3, b3):
    f32 = jnp.float32
    fp = jnp.zeros((NP, K1), f32).at[:N, :1433].set(features)
    W1p = jnp.zeros((K1, H1), f32).at[:1433, :1000].set(W1)
    b1p = jnp.zeros((H1 // W, 1, W), f32).at[:, 0, :].set(
        jnp.pad(b1, (0, H1 - 1000)).reshape(H1 // W, W))
    W2p = jnp.zeros((H1, H2), f32).at[:1000, :500].set(W2)
    b2p = jnp.zeros((H2 // W, 1, W), f32).at[:, 0, :].set(
        jnp.pad(b2, (0, H2 - 500)).reshape(H2 // W, W))
    W3p = jnp.zeros((H2, DO), f32).at[:500, :].set(W3)
    b3p = b3.reshape(1, DO)

    pad = jnp.full((EP - E,), N, jnp.int32)
    srcp = jnp.concatenate([edge_index[0], pad])
    dstp = jnp.concatenate([edge_index[1], pad])
    zeros = jnp.zeros((NP, W), f32)

    h1 = _mm1(fp, W1p)
    agg1 = _agg8(h1, srcp, dstp, zeros)
    h2 = _mm2(agg1, b1p, W2p)
    agg2 = _agg4(h2, srcp, dstp, zeros)
    out = _mm3(agg2, b2p, W3p, b3p)
    return out[:N]


# speedup vs baseline: 2.3034x; 1.8775x over previous
import functools

import jax
import jax.numpy as jnp
from jax import lax
from jax.experimental import pallas as pl
from jax.experimental.pallas import tpu as pltpu
from jax.experimental.pallas import tpu_sc as plsc

N = 10000
NP = 10240
E = 50000
EB = 128
NB = 25
EP = 16 * NB * EB
MB = 400
K1 = 1433
H1 = 1024
H2 = 512
W = 128
DO = 128


def _mm1_body(a_ref, w_ref, o_ref):
    o_ref[0] = jnp.dot(a_ref[...], w_ref[...],
                       preferred_element_type=jnp.float32)


def _mm1(a, w):
    c = H1 // W
    return pl.pallas_call(
        _mm1_body,
        grid=(N // MB, c),
        in_specs=[
            pl.BlockSpec((MB, K1), lambda i, j: (i, 0)),
            pl.BlockSpec((K1, W), lambda i, j: (0, j)),
        ],
        out_specs=pl.BlockSpec((1, MB, W), lambda i, j: (j, i, 0)),
        out_shape=jax.ShapeDtypeStruct((c, N, W), jnp.float32),
    )(a, w)


def _mm2_body(a_ref, b_ref, w_ref, o_ref):
    k = pl.program_id(2)
    x = jnp.maximum(a_ref[0] + b_ref[0], 0.0)
    y = jnp.dot(x, w_ref[...], preferred_element_type=jnp.float32)

    @pl.when(k == 0)
    def _():
        o_ref[0] = y

    @pl.when(k > 0)
    def _():
        o_ref[0] += y


def _mm2(agg, b, w):
    ci = agg.shape[0]
    co = w.shape[1] // W
    return pl.pallas_call(
        _mm2_body,
        grid=(N // MB, co, ci),
        in_specs=[
            pl.BlockSpec((1, MB, W), lambda i, j, k: (k, i, 0)),
            pl.BlockSpec((1, 1, W), lambda i, j, k: (k, 0, 0)),
            pl.BlockSpec((W, W), lambda i, j, k: (k, j)),
        ],
        out_specs=pl.BlockSpec((1, MB, W), lambda i, j, k: (j, i, 0)),
        out_shape=jax.ShapeDtypeStruct((co, N, W), jnp.float32),
        compiler_params=pltpu.CompilerParams(
            dimension_semantics=("parallel", "parallel", "arbitrary")),
    )(agg, b, w)


def _mm3_body(a_ref, b2_ref, w_ref, b3_ref, o_ref, acc_ref):
    k = pl.program_id(1)
    x = jnp.maximum(a_ref[0] + b2_ref[0], 0.0)
    y = jnp.dot(x, w_ref[...], preferred_element_type=jnp.float32)

    @pl.when(k == 0)
    def _():
        acc_ref[...] = y

    @pl.when(k > 0)
    def _():
        acc_ref[...] += y

    @pl.when(k == pl.num_programs(1) - 1)
    def _():
        o_ref[...] = jnp.maximum(acc_ref[...] + b3_ref[...], 0.0)


def _mm3(agg, b2, w, b3):
    ci = agg.shape[0]
    return pl.pallas_call(
        _mm3_body,
        grid=(N // MB, ci),
        in_specs=[
            pl.BlockSpec((1, MB, W), lambda i, k: (k, i, 0)),
            pl.BlockSpec((1, 1, W), lambda i, k: (k, 0, 0)),
            pl.BlockSpec((W, DO), lambda i, k: (k, 0)),
            pl.BlockSpec((1, DO), lambda i, k: (0, 0)),
        ],
        out_specs=pl.BlockSpec((MB, DO), lambda i, k: (i, 0)),
        out_shape=jax.ShapeDtypeStruct((N, DO), jnp.float32),
        scratch_shapes=[pltpu.VMEM((MB, DO), jnp.float32)],
        compiler_params=pltpu.CompilerParams(
            dimension_semantics=("parallel", "arbitrary")),
    )(agg, b2, w, b3)


def _make_agg(c_chunks):
    half = c_chunks // 2
    rows_t = NP // 16
    mesh = plsc.VectorSubcoreMesh(core_axis_name="c", subcore_axis_name="s")

    @functools.partial(
        pl.kernel,
        mesh=mesh,
        out_type=jax.ShapeDtypeStruct((c_chunks, NP, W), jnp.float32),
        scratch_types=[
            pltpu.VMEM_SHARED((NP, W), jnp.float32),
            pltpu.VMEM((NB, EB), jnp.int32),
            pltpu.VMEM((NB, EB), jnp.int32),
            pltpu.VMEM((EB, W), jnp.float32),
            pltpu.SemaphoreType.DMA,
        ],
    )
    def agg(h, src, dst, zeros, out, slab, src_v, dst_v, rows_v, sem):
        cid = lax.axis_index("c")
        sid = lax.axis_index("s")
        r0 = sid * rows_t
        pltpu.sync_copy(src.at[sid], src_v)
        pltpu.sync_copy(dst.at[sid], dst_v)

        def chunk_body(ci, carry):
            chunk = cid * half + ci
            pltpu.sync_copy(zeros.at[pl.ds(r0, rows_t)],
                            slab.at[pl.ds(r0, rows_t)])
            plsc.subcore_barrier()

            def batch_body(b, carry2):
                pltpu.async_copy(h.at[chunk].at[src_v.at[b]], rows_v,
                                 sem).wait()
                pltpu.sync_copy(rows_v, slab.at[dst_v.at[b]], add=True)
                return carry2

            lax.fori_loop(0, NB, batch_body, 0)
            plsc.subcore_barrier()
            pltpu.sync_copy(slab.at[pl.ds(r0, rows_t)],
                            out.at[chunk, pl.ds(r0, rows_t)])
            plsc.subcore_barrier()
            return carry

        lax.fori_loop(0, half, chunk_body, 0)

    return agg


_agg8 = _make_agg(8)
_agg4 = _make_agg(4)


def kernel(features, edge_index, W1, b1, W2, b2, W3, b3):
    f32 = jnp.float32
    W1p = jnp.zeros((K1, H1), f32).at[:, :1000].set(W1)
    b1p = jnp.zeros((H1 // W, 1, W), f32).at[:, 0, :].set(
        jnp.pad(b1, (0, H1 - 1000)).reshape(H1 // W, W))
    W2p = jnp.zeros((H1, H2), f32).at[:1000, :500].set(W2)
    b2p = jnp.zeros((H2 // W, 1, W), f32).at[:, 0, :].set(
        jnp.pad(b2, (0, H2 - 500)).reshape(H2 // W, W))
    W3p = jnp.zeros((H2, DO), f32).at[:500, :].set(W3)
    b3p = b3.reshape(1, DO)

    npad = EP - E
    pad_i = jnp.arange(npad, dtype=jnp.int32)
    srcp = jnp.concatenate([edge_index[0], pad_i % N]).reshape(16, NB, EB)
    dstp = jnp.concatenate([edge_index[1], N + pad_i % (NP - N)]
                           ).reshape(16, NB, EB)
    zeros = jnp.zeros((NP, W), f32)

    h1 = _mm1(features, W1p)
    agg1 = _agg8(h1, srcp, dstp, zeros)
    h2 = _mm2(agg1, b1p, W2p)
    agg2 = _agg4(h2, srcp, dstp, zeros)
    return _mm3(agg2, b2p, W3p, b3p)


# speedup vs baseline: 2.3140x; 1.0046x over previous
import functools

import jax
import jax.numpy as jnp
from jax import lax
from jax.experimental import pallas as pl
from jax.experimental.pallas import tpu as pltpu
from jax.experimental.pallas import tpu_sc as plsc

N = 10000
NP = 10240
E = 50000
EB = 128
NB = 25
EP = 16 * NB * EB
MB = 400
K1 = 1433
H1 = 1024
H2 = 512
W = 128
DO = 128


def _mm1_body(a_ref, w_ref, o_ref):
    o_ref[0] = jnp.dot(a_ref[...], w_ref[...],
                       preferred_element_type=jnp.float32)


def _bf(x):
    return x.astype(jnp.bfloat16)


def _mm1(a, w):
    c = H1 // W
    return pl.pallas_call(
        _mm1_body,
        grid=(N // MB, c),
        in_specs=[
            pl.BlockSpec((MB, K1), lambda i, j: (i, 0)),
            pl.BlockSpec((K1, W), lambda i, j: (0, j)),
        ],
        out_specs=pl.BlockSpec((1, MB, W), lambda i, j: (j, i, 0)),
        out_shape=jax.ShapeDtypeStruct((c, N, W), jnp.float32),
    )(a, w)


def _mm2_body(a_ref, b_ref, w_ref, o_ref):
    k = pl.program_id(2)
    x = jnp.maximum(a_ref[0] + b_ref[0], 0.0)
    y = jnp.dot(_bf(x), w_ref[...], preferred_element_type=jnp.float32)

    @pl.when(k == 0)
    def _():
        o_ref[0] = y

    @pl.when(k > 0)
    def _():
        o_ref[0] += y


def _mm2(agg, b, w):
    ci = agg.shape[0]
    co = w.shape[1] // W
    return pl.pallas_call(
        _mm2_body,
        grid=(N // MB, co, ci),
        in_specs=[
            pl.BlockSpec((1, MB, W), lambda i, j, k: (k, i, 0)),
            pl.BlockSpec((1, 1, W), lambda i, j, k: (k, 0, 0)),
            pl.BlockSpec((W, W), lambda i, j, k: (k, j)),
        ],
        out_specs=pl.BlockSpec((1, MB, W), lambda i, j, k: (j, i, 0)),
        out_shape=jax.ShapeDtypeStruct((co, N, W), jnp.float32),
        compiler_params=pltpu.CompilerParams(
            dimension_semantics=("parallel", "parallel", "arbitrary")),
    )(agg, b, w)


def _mm3_body(a_ref, b2_ref, w_ref, b3_ref, o_ref, acc_ref):
    k = pl.program_id(1)
    x = jnp.maximum(a_ref[0] + b2_ref[0], 0.0)
    y = jnp.dot(_bf(x), w_ref[...], preferred_element_type=jnp.float32)

    @pl.when(k == 0)
    def _():
        acc_ref[...] = y

    @pl.when(k > 0)
    def _():
        acc_ref[...] += y

    @pl.when(k == pl.num_programs(1) - 1)
    def _():
        o_ref[...] = jnp.maximum(acc_ref[...] + b3_ref[...], 0.0)


def _mm3(agg, b2, w, b3):
    ci = agg.shape[0]
    return pl.pallas_call(
        _mm3_body,
        grid=(N // MB, ci),
        in_specs=[
            pl.BlockSpec((1, MB, W), lambda i, k: (k, i, 0)),
            pl.BlockSpec((1, 1, W), lambda i, k: (k, 0, 0)),
            pl.BlockSpec((W, DO), lambda i, k: (k, 0)),
            pl.BlockSpec((1, DO), lambda i, k: (0, 0)),
        ],
        out_specs=pl.BlockSpec((MB, DO), lambda i, k: (i, 0)),
        out_shape=jax.ShapeDtypeStruct((N, DO), jnp.float32),
        scratch_shapes=[pltpu.VMEM((MB, DO), jnp.float32)],
        compiler_params=pltpu.CompilerParams(
            dimension_semantics=("parallel", "arbitrary")),
    )(agg, b2, w, b3)


def _make_agg(c_chunks):
    half = c_chunks // 2
    rows_t = NP // 16
    mesh = plsc.VectorSubcoreMesh(core_axis_name="c", subcore_axis_name="s")

    @functools.partial(
        pl.kernel,
        mesh=mesh,
        out_type=jax.ShapeDtypeStruct((c_chunks, NP, W), jnp.float32),
        scratch_types=[
            pltpu.VMEM_SHARED((NP, W), jnp.float32),
            pltpu.VMEM((NB, EB), jnp.int32),
            pltpu.VMEM((NB, EB), jnp.int32),
            pltpu.VMEM((EB, W), jnp.float32),
            pltpu.SemaphoreType.DMA,
        ],
    )
    def agg(h, src, dst, zeros, out, slab, src_v, dst_v, rows_v, sem):
        cid = lax.axis_index("c")
        sid = lax.axis_index("s")
        r0 = sid * rows_t
        pltpu.sync_copy(src.at[sid], src_v)
        pltpu.sync_copy(dst.at[sid], dst_v)

        def chunk_body(ci, carry):
            chunk = cid * half + ci
            pltpu.sync_copy(zeros.at[pl.ds(r0, rows_t)],
                            slab.at[pl.ds(r0, rows_t)])
            plsc.subcore_barrier()

            def batch_body(b, carry2):
                pltpu.async_copy(h.at[chunk].at[src_v.at[b]], rows_v,
                                 sem).wait()
                pltpu.sync_copy(rows_v, slab.at[dst_v.at[b]], add=True)
                return carry2

            lax.fori_loop(0, NB, batch_body, 0)
            plsc.subcore_barrier()
            pltpu.sync_copy(slab.at[pl.ds(r0, rows_t)],
                            out.at[chunk, pl.ds(r0, rows_t)])
            plsc.subcore_barrier()
            return carry

        lax.fori_loop(0, half, chunk_body, 0)

    return agg


_agg8 = _make_agg(8)
_agg4 = _make_agg(4)


def kernel(features, edge_index, W1, b1, W2, b2, W3, b3):
    f32 = jnp.float32
    W1p = jnp.zeros((K1, H1), f32).at[:, :1000].set(W1)
    b1p = jnp.zeros((H1 // W, 1, W), f32).at[:, 0, :].set(
        jnp.pad(b1, (0, H1 - 1000)).reshape(H1 // W, W))
    W2p = jnp.zeros((H1, H2), f32).at[:1000, :500].set(W2)
    b2p = jnp.zeros((H2 // W, 1, W), f32).at[:, 0, :].set(
        jnp.pad(b2, (0, H2 - 500)).reshape(H2 // W, W))
    W3p = jnp.zeros((H2, DO), f32).at[:500, :].set(W3)
    b3p = b3.reshape(1, DO)

    npad = EP - E
    pad_i = jnp.arange(npad, dtype=jnp.int32)
    srcp = jnp.concatenate([edge_index[0], pad_i % N]).reshape(16, NB, EB)
    dstp = jnp.concatenate([edge_index[1], N + pad_i % (NP - N)]
                           ).reshape(16, NB, EB)
    zeros = jnp.zeros((NP, W), f32)

    h1 = _mm1(_bf(features), _bf(W1p))
    agg1 = _agg8(h1, srcp, dstp, zeros)
    h2 = _mm2(agg1, b1p, _bf(W2p))
    agg2 = _agg4(h2, srcp, dstp, zeros)
    return _mm3(agg2, b2p, _bf(W3p), b3p)


# speedup vs baseline: 4.8990x; 2.1172x over previous
import functools

import jax
import jax.numpy as jnp
from jax import lax
from jax.experimental import pallas as pl
from jax.experimental.pallas import tpu as pltpu
from jax.experimental.pallas import tpu_sc as plsc

N = 10000
NP = 10240
E = 50000
EB = 128
NB = 25
EP = 16 * NB * EB
MB = 400
K1 = 1433
H1 = 1024
H2 = 512
W = 128
DO = 128


def _bf(x):
    return x.astype(jnp.bfloat16)


def _mm1_body(a_ref, w_ref, o_ref):
    y = jnp.dot(_bf(a_ref[...]), w_ref[...],
                preferred_element_type=jnp.float32)
    for c in range(H1 // W):
        o_ref[c] = y[:, c * W:(c + 1) * W]


def _mm1(a, w):
    c = H1 // W
    return pl.pallas_call(
        _mm1_body,
        grid=(N // MB,),
        in_specs=[
            pl.BlockSpec((MB, K1), lambda i: (i, 0)),
            pl.BlockSpec((K1, H1), lambda i: (0, 0)),
        ],
        out_specs=pl.BlockSpec((c, MB, W), lambda i: (0, i, 0)),
        out_shape=jax.ShapeDtypeStruct((c, N, W), jnp.float32),
    )(a, w)


def _mm2_body(a_ref, b_ref, w_ref, o_ref):
    x = jnp.maximum(a_ref[...] + b_ref[...], 0.0)
    y = jnp.dot(_bf(x), w_ref[...], preferred_element_type=jnp.float32)
    for c in range(H2 // W):
        o_ref[c] = y[:, c * W:(c + 1) * W]


def _mm2(agg, b, w):
    c = H2 // W
    return pl.pallas_call(
        _mm2_body,
        grid=(N // MB,),
        in_specs=[
            pl.BlockSpec((MB, H1), lambda i: (i, 0)),
            pl.BlockSpec((1, H1), lambda i: (0, 0)),
            pl.BlockSpec((H1, H2), lambda i: (0, 0)),
        ],
        out_specs=pl.BlockSpec((c, MB, W), lambda i: (0, i, 0)),
        out_shape=jax.ShapeDtypeStruct((c, N, W), jnp.float32),
    )(agg, b, w)


def _mm3_body(a_ref, b2_ref, w_ref, b3_ref, o_ref):
    x = jnp.maximum(a_ref[...] + b2_ref[...], 0.0)
    y = jnp.dot(_bf(x), w_ref[...], preferred_element_type=jnp.float32)
    o_ref[...] = jnp.maximum(y + b3_ref[...], 0.0)


def _mm3(agg, b2, w, b3):
    return pl.pallas_call(
        _mm3_body,
        grid=(N // MB,),
        in_specs=[
            pl.BlockSpec((MB, H2), lambda i: (i, 0)),
            pl.BlockSpec((1, H2), lambda i: (0, 0)),
            pl.BlockSpec((H2, DO), lambda i: (0, 0)),
            pl.BlockSpec((1, DO), lambda i: (0, 0)),
        ],
        out_specs=pl.BlockSpec((MB, DO), lambda i: (i, 0)),
        out_shape=jax.ShapeDtypeStruct((N, DO), jnp.float32),
    )(agg, b2, w, b3)


def _make_agg(c_chunks):
    half = c_chunks // 2
    rows_t = NP // 16
    mesh = plsc.VectorSubcoreMesh(core_axis_name="c", subcore_axis_name="s")

    @functools.partial(
        pl.kernel,
        mesh=mesh,
        out_type=jax.ShapeDtypeStruct((NP, c_chunks * W), jnp.float32),
        scratch_types=[
            pltpu.VMEM_SHARED((NP, W), jnp.float32),
            pltpu.VMEM((NB, EB), jnp.int32),
            pltpu.VMEM((NB, EB), jnp.int32),
            pltpu.VMEM((EB, W), jnp.float32),
            pltpu.SemaphoreType.DMA,
        ],
    )
    def agg(h, src, dst, zeros, out, slab, src_v, dst_v, rows_v, sem):
        cid = lax.axis_index("c")
        sid = lax.axis_index("s")
        r0 = sid * rows_t
        pltpu.sync_copy(src.at[sid], src_v)
        pltpu.sync_copy(dst.at[sid], dst_v)

        def chunk_body(ci, carry):
            chunk = cid * half + ci
            pltpu.sync_copy(zeros.at[pl.ds(r0, rows_t)],
                            slab.at[pl.ds(r0, rows_t)])
            plsc.subcore_barrier()

            def batch_body(b, carry2):
                pltpu.async_copy(h.at[chunk].at[src_v.at[b]], rows_v,
                                 sem).wait()
                pltpu.sync_copy(rows_v, slab.at[dst_v.at[b]], add=True)
                return carry2

            lax.fori_loop(0, NB, batch_body, 0)
            plsc.subcore_barrier()
            pltpu.sync_copy(slab.at[pl.ds(r0, rows_t)],
                            out.at[pl.ds(r0, rows_t), pl.ds(chunk * W, W)])
            plsc.subcore_barrier()
            return carry

        lax.fori_loop(0, half, chunk_body, 0)

    return agg


_agg8 = _make_agg(8)
_agg4 = _make_agg(4)


def kernel(features, edge_index, W1, b1, W2, b2, W3, b3):
    f32 = jnp.float32
    bf16 = jnp.bfloat16
    W1p = jnp.zeros((K1, H1), bf16).at[:, :1000].set(_bf(W1))
    b1p = jnp.pad(b1, (0, H1 - 1000)).reshape(1, H1)
    W2p = jnp.zeros((H1, H2), bf16).at[:1000, :500].set(_bf(W2))
    b2p = jnp.pad(b2, (0, H2 - 500)).reshape(1, H2)
    W3p = jnp.zeros((H2, DO), bf16).at[:500, :].set(_bf(W3))
    b3p = b3.reshape(1, DO)

    npad = EP - E
    pad_i = jnp.arange(npad, dtype=jnp.int32)
    srcp = jnp.concatenate([edge_index[0], pad_i % N]).reshape(16, NB, EB)
    dstp = jnp.concatenate([edge_index[1], N + pad_i % (NP - N)]
                           ).reshape(16, NB, EB)
    zeros = jnp.zeros((NP, W), f32)

    h1 = _mm1(features, W1p)
    agg1 = _agg8(h1, srcp, dstp, zeros)
    h2 = _mm2(agg1, b1p, W2p)
    agg2 = _agg4(h2, srcp, dstp, zeros)
    return _mm3(agg2, b2p, W3p, b3p)


# speedup vs baseline: 6.1513x; 1.2556x over previous
import functools

import jax
import jax.numpy as jnp
from jax import lax
from jax.experimental import pallas as pl
from jax.experimental.pallas import tpu as pltpu
from jax.experimental.pallas import tpu_sc as plsc

N = 10000
NP = 10240
E = 50000
EB = 128
NB = 26
EP = 16 * NB * EB
MB = 400
K1 = 1433
H1 = 1024
H2 = 512
W = 128
DO = 128


def _bf(x):
    return x.astype(jnp.bfloat16)


def _mm1_body(a_ref, w_ref, o_ref):
    y = jnp.dot(_bf(a_ref[...]), w_ref[...],
                preferred_element_type=jnp.float32)
    for c in range(H1 // W):
        o_ref[c] = y[:, c * W:(c + 1) * W]


def _mm1(a, w):
    c = H1 // W
    return pl.pallas_call(
        _mm1_body,
        grid=(N // MB,),
        in_specs=[
            pl.BlockSpec((MB, K1), lambda i: (i, 0)),
            pl.BlockSpec((K1, H1), lambda i: (0, 0)),
        ],
        out_specs=pl.BlockSpec((c, MB, W), lambda i: (0, i, 0)),
        out_shape=jax.ShapeDtypeStruct((c, N, W), jnp.float32),
    )(a, w)


def _mm2_body(a_ref, b_ref, w_ref, o_ref):
    x = jnp.maximum(a_ref[...] + b_ref[...], 0.0)
    y = jnp.dot(_bf(x), w_ref[...], preferred_element_type=jnp.float32)
    for c in range(H2 // W):
        o_ref[c] = y[:, c * W:(c + 1) * W]


def _mm2(agg, b, w):
    c = H2 // W
    return pl.pallas_call(
        _mm2_body,
        grid=(N // MB,),
        in_specs=[
            pl.BlockSpec((MB, H1), lambda i: (i, 0)),
            pl.BlockSpec((1, H1), lambda i: (0, 0)),
            pl.BlockSpec((H1, H2), lambda i: (0, 0)),
        ],
        out_specs=pl.BlockSpec((c, MB, W), lambda i: (0, i, 0)),
        out_shape=jax.ShapeDtypeStruct((c, N, W), jnp.float32),
    )(agg, b, w)


def _mm3_body(a_ref, b2_ref, w_ref, b3_ref, o_ref):
    x = jnp.maximum(a_ref[...] + b2_ref[...], 0.0)
    y = jnp.dot(_bf(x), w_ref[...], preferred_element_type=jnp.float32)
    o_ref[...] = jnp.maximum(y + b3_ref[...], 0.0)


def _mm3(agg, b2, w, b3):
    return pl.pallas_call(
        _mm3_body,
        grid=(N // MB,),
        in_specs=[
            pl.BlockSpec((MB, H2), lambda i: (i, 0)),
            pl.BlockSpec((1, H2), lambda i: (0, 0)),
            pl.BlockSpec((H2, DO), lambda i: (0, 0)),
            pl.BlockSpec((1, DO), lambda i: (0, 0)),
        ],
        out_specs=pl.BlockSpec((MB, DO), lambda i: (i, 0)),
        out_shape=jax.ShapeDtypeStruct((N, DO), jnp.float32),
    )(agg, b2, w, b3)


def _make_agg(c_chunks):
    half = c_chunks // 2
    rows_t = NP // 16
    mesh = plsc.VectorSubcoreMesh(core_axis_name="c", subcore_axis_name="s")

    @functools.partial(
        pl.kernel,
        mesh=mesh,
        out_type=jax.ShapeDtypeStruct((NP, c_chunks * W), jnp.float32),
        scratch_types=[
            pltpu.VMEM_SHARED((NP, W), jnp.float32),
            pltpu.VMEM((NB, EB), jnp.int32),
            pltpu.VMEM((NB, EB), jnp.int32),
            pltpu.VMEM((EB, W), jnp.float32),
            pltpu.VMEM((EB, W), jnp.float32),
            pltpu.SemaphoreType.DMA,
            pltpu.SemaphoreType.DMA,
        ],
    )
    def agg(h, src, dst, zeros, out, slab, src_v, dst_v, rows0, rows1,
            sem0, sem1):
        bufs = (rows0, rows1)
        sems = (sem0, sem1)
        cid = lax.axis_index("c")
        sid = lax.axis_index("s")
        r0 = sid * rows_t
        pltpu.sync_copy(src.at[sid], src_v)
        pltpu.sync_copy(dst.at[sid], dst_v)

        def chunk_body(ci, carry):
            chunk = cid * half + ci
            pltpu.sync_copy(zeros.at[pl.ds(r0, rows_t)],
                            slab.at[pl.ds(r0, rows_t)])
            plsc.subcore_barrier()

            pltpu.async_copy(h.at[chunk].at[src_v.at[0]], bufs[0], sems[0])

            def batch_body(o, carry2):
                for j in range(2):
                    b = o + j

                    @pl.when(b + 1 < NB)
                    def _():
                        pltpu.async_copy(h.at[chunk].at[src_v.at[b + 1]],
                                         bufs[1 - j], sems[1 - j])

                    pltpu.make_async_copy(h.at[chunk].at[src_v.at[b]],
                                          bufs[j], sems[j]).wait()
                    pltpu.sync_copy(bufs[j], slab.at[dst_v.at[b]], add=True)
                return carry2

            lax.fori_loop(0, NB // 2, lambda o, c: batch_body(o * 2, c), 0)
            plsc.subcore_barrier()
            pltpu.sync_copy(slab.at[pl.ds(r0, rows_t)],
                            out.at[pl.ds(r0, rows_t), pl.ds(chunk * W, W)])
            plsc.subcore_barrier()
            return carry

        lax.fori_loop(0, half, chunk_body, 0)

    return agg


_agg8 = _make_agg(8)
_agg4 = _make_agg(4)


def kernel(features, edge_index, W1, b1, W2, b2, W3, b3):
    f32 = jnp.float32
    bf16 = jnp.bfloat16
    W1p = jnp.zeros((K1, H1), bf16).at[:, :1000].set(_bf(W1))
    b1p = jnp.pad(b1, (0, H1 - 1000)).reshape(1, H1)
    W2p = jnp.zeros((H1, H2), bf16).at[:1000, :500].set(_bf(W2))
    b2p = jnp.pad(b2, (0, H2 - 500)).reshape(1, H2)
    W3p = jnp.zeros((H2, DO), bf16).at[:500, :].set(_bf(W3))
    b3p = b3.reshape(1, DO)

    npad = EP - E
    pad_i = jnp.arange(npad, dtype=jnp.int32)
    srcp = jnp.concatenate([edge_index[0], pad_i % N]).reshape(16, NB, EB)
    dstp = jnp.concatenate([edge_index[1], N + pad_i % (NP - N)]
                           ).reshape(16, NB, EB)
    zeros = jnp.zeros((NP, W), f32)

    h1 = _mm1(features, W1p)
    agg1 = _agg8(h1, srcp, dstp, zeros)
    h2 = _mm2(agg1, b1p, W2p)
    agg2 = _agg4(h2, srcp, dstp, zeros)
    return _mm3(agg2, b2p, W3p, b3p)


# speedup vs baseline: 6.9881x; 1.1360x over previous
import functools

import jax
import jax.numpy as jnp
from jax import lax
from jax.experimental import pallas as pl
from jax.experimental.pallas import tpu as pltpu
from jax.experimental.pallas import tpu_sc as plsc

N = 10000
NP = 10240
E = 50000
EB = 128
NB = 26
EP = 16 * NB * EB
MB = 400
K1 = 1433
H1 = 1024
H2 = 512
W = 128
DO = 128


def _bf(x):
    return x.astype(jnp.bfloat16)


def _mm1_body(a_ref, w_ref, o_ref):
    y = lax.dot_general(_bf(a_ref[...]), w_ref[...],
                        dimension_numbers=(((0,), (0,)), ((), ())),
                        preferred_element_type=jnp.float32)
    for c in range(H1 // W):
        o_ref[c] = y[:, c * W:(c + 1) * W]


def _mm1(at, w):
    c = H1 // W
    mb = 512
    return pl.pallas_call(
        _mm1_body,
        grid=(pl.cdiv(N, mb),),
        in_specs=[
            pl.BlockSpec((K1, mb), lambda i: (0, i)),
            pl.BlockSpec((K1, H1), lambda i: (0, 0)),
        ],
        out_specs=pl.BlockSpec((c, mb, W), lambda i: (0, i, 0)),
        out_shape=jax.ShapeDtypeStruct((c, N, W), jnp.float32),
    )(at, w)


def _mm2_body(a_ref, b_ref, w_ref, o_ref):
    x = jnp.maximum(a_ref[...] + b_ref[...], 0.0)
    y = jnp.dot(_bf(x), w_ref[...], preferred_element_type=jnp.float32)
    for c in range(H2 // W):
        o_ref[c] = y[:, c * W:(c + 1) * W]


def _mm2(agg, b, w):
    c = H2 // W
    return pl.pallas_call(
        _mm2_body,
        grid=(N // MB,),
        in_specs=[
            pl.BlockSpec((MB, H1), lambda i: (i, 0)),
            pl.BlockSpec((1, H1), lambda i: (0, 0)),
            pl.BlockSpec((H1, H2), lambda i: (0, 0)),
        ],
        out_specs=pl.BlockSpec((c, MB, W), lambda i: (0, i, 0)),
        out_shape=jax.ShapeDtypeStruct((c, N, W), jnp.float32),
    )(agg, b, w)


def _mm3_body(a_ref, b2_ref, w_ref, b3_ref, o_ref):
    x = jnp.maximum(a_ref[...] + b2_ref[...], 0.0)
    y = jnp.dot(_bf(x), w_ref[...], preferred_element_type=jnp.float32)
    o_ref[...] = jnp.maximum(y + b3_ref[...], 0.0)


def _mm3(agg, b2, w, b3):
    return pl.pallas_call(
        _mm3_body,
        grid=(N // MB,),
        in_specs=[
            pl.BlockSpec((MB, H2), lambda i: (i, 0)),
            pl.BlockSpec((1, H2), lambda i: (0, 0)),
            pl.BlockSpec((H2, DO), lambda i: (0, 0)),
            pl.BlockSpec((1, DO), lambda i: (0, 0)),
        ],
        out_specs=pl.BlockSpec((MB, DO), lambda i: (i, 0)),
        out_shape=jax.ShapeDtypeStruct((N, DO), jnp.float32),
    )(agg, b2, w, b3)


def _make_agg(c_chunks):
    half = c_chunks // 2
    rows_t = NP // 16
    mesh = plsc.VectorSubcoreMesh(core_axis_name="c", subcore_axis_name="s")

    @functools.partial(
        pl.kernel,
        mesh=mesh,
        out_type=jax.ShapeDtypeStruct((NP, c_chunks * W), jnp.float32),
        scratch_types=[
            pltpu.VMEM_SHARED((NP, W), jnp.float32),
            pltpu.VMEM((NB, EB), jnp.int32),
            pltpu.VMEM((NB, EB), jnp.int32),
            pltpu.VMEM((EB, W), jnp.float32),
            pltpu.VMEM((EB, W), jnp.float32),
            pltpu.SemaphoreType.DMA,
            pltpu.SemaphoreType.DMA,
        ],
    )
    def agg(h, src, dst, zeros, out, slab, src_v, dst_v, rows0, rows1,
            sem0, sem1):
        bufs = (rows0, rows1)
        sems = (sem0, sem1)
        cid = lax.axis_index("c")
        sid = lax.axis_index("s")
        r0 = sid * rows_t
        pltpu.sync_copy(src.at[sid], src_v)
        pltpu.sync_copy(dst.at[sid], dst_v)

        def chunk_body(ci, carry):
            chunk = cid * half + ci
            pltpu.sync_copy(zeros.at[pl.ds(r0, rows_t)],
                            slab.at[pl.ds(r0, rows_t)])
            plsc.subcore_barrier()

            pltpu.async_copy(h.at[chunk].at[src_v.at[0]], bufs[0], sems[0])

            def batch_body(o, carry2):
                for j in range(2):
                    b = o + j

                    @pl.when(b + 1 < NB)
                    def _():
                        pltpu.async_copy(h.at[chunk].at[src_v.at[b + 1]],
                                         bufs[1 - j], sems[1 - j])

                    pltpu.make_async_copy(h.at[chunk].at[src_v.at[b]],
                                          bufs[j], sems[j]).wait()
                    pltpu.sync_copy(bufs[j], slab.at[dst_v.at[b]], add=True)
                return carry2

            lax.fori_loop(0, NB // 2, lambda o, c: batch_body(o * 2, c), 0)
            plsc.subcore_barrier()
            pltpu.sync_copy(slab.at[pl.ds(r0, rows_t)],
                            out.at[pl.ds(r0, rows_t), pl.ds(chunk * W, W)])
            plsc.subcore_barrier()
            return carry

        lax.fori_loop(0, half, chunk_body, 0)

    return agg


_agg8 = _make_agg(8)
_agg4 = _make_agg(4)


def kernel(features, edge_index, W1, b1, W2, b2, W3, b3):
    f32 = jnp.float32
    bf16 = jnp.bfloat16
    W1p = jnp.zeros((K1, H1), bf16).at[:, :1000].set(_bf(W1))
    b1p = jnp.pad(b1, (0, H1 - 1000)).reshape(1, H1)
    W2p = jnp.zeros((H1, H2), bf16).at[:1000, :500].set(_bf(W2))
    b2p = jnp.pad(b2, (0, H2 - 500)).reshape(1, H2)
    W3p = jnp.zeros((H2, DO), bf16).at[:500, :].set(_bf(W3))
    b3p = b3.reshape(1, DO)

    npad = EP - E
    pad_i = jnp.arange(npad, dtype=jnp.int32)
    srcp = jnp.concatenate([edge_index[0], pad_i % N]).reshape(16, NB, EB)
    dstp = jnp.concatenate([edge_index[1], N + pad_i % (NP - N)]
                           ).reshape(16, NB, EB)
    zeros = jnp.zeros((NP, W), f32)

    h1 = _mm1(features.T, W1p)
    agg1 = _agg8(h1, srcp, dstp, zeros)
    h2 = _mm2(agg1, b1p, W2p)
    agg2 = _agg4(h2, srcp, dstp, zeros)
    return _mm3(agg2, b2p, W3p, b3p)
